# baseline (speedup 1.0000x reference)
import functools

import jax
import jax.numpy as jnp
from jax import lax
from jax.experimental import pallas as pl
from jax.experimental.pallas import tpu as pltpu
from jax.experimental.pallas import tpu_sc as plsc

B, S, D = 4096, 77, 512
N = B * S
NC, NS = 2, 16
NW = NC * NS
RPW = N // NW
K = 128
NCHUNK = RPW // K
LANES = 16


def _emb_body(x_hbm, tok_hbm, pos_hbm, out_hbm, idx_v, pos_v, rows_v, sem_g):
    wid = lax.axis_index("s") * NC + lax.axis_index("c")
    base = wid * RPW

    pltpu.sync_copy(x_hbm.at[pl.ds(base, RPW)], idx_v)
    pltpu.sync_copy(pos_hbm, pos_v)

    def chunk(c, s0):
        pltpu.async_copy(
            tok_hbm.at[idx_v.at[pl.ds(c * K, K)]], rows_v, sem_g
        ).wait()

        def radd(r, s):
            for j in range(D // LANES):
                sl = pl.ds(j * LANES, LANES)
                rows_v[r, sl] = rows_v[r, sl] + pos_v[s, sl]
            return lax.select(s == S - 1, 0, s + 1)

        s_next = lax.fori_loop(0, K, radd, s0)
        pltpu.sync_copy(rows_v, out_hbm.at[pl.ds(base + c * K, K)])
        return s_next

    lax.fori_loop(0, NCHUNK, chunk, 0)


@functools.partial(
    pl.kernel,
    out_type=jax.ShapeDtypeStruct((N, D), jnp.float32),
    mesh=plsc.VectorSubcoreMesh(
        core_axis_name="c", subcore_axis_name="s", num_cores=NC, num_subcores=NS
    ),
    scratch_types=[
        pltpu.VMEM((RPW,), jnp.int32),
        pltpu.VMEM((S, D), jnp.float32),
        pltpu.VMEM((K, D), jnp.float32),
        pltpu.SemaphoreType.DMA,
    ],
)
def _emb(x_hbm, tok_hbm, pos_hbm, out_hbm, idx_v, pos_v, rows_v, sem_g):
    _emb_body(x_hbm, tok_hbm, pos_hbm, out_hbm, idx_v, pos_v, rows_v, sem_g)


def kernel(x, token_table, position_table):
    out = _emb(x.astype(jnp.int32).reshape(N), token_table, position_table)
    return out.reshape(B, S, D)


# speedup vs baseline: 1.1009x; 1.1009x over previous
import functools

import jax
import jax.numpy as jnp
from jax import lax
from jax.experimental import pallas as pl
from jax.experimental.pallas import tpu as pltpu
from jax.experimental.pallas import tpu_sc as plsc

B, S, D = 4096, 77, 512
N = B * S
NC, NS = 2, 16
NW = NC * NS
RPW = N // NW
K = 64
NCHUNK = RPW // K
LANES = 16


def _emb_body(x_hbm, tok_hbm, pos_hbm, out_hbm,
              idx_v, pos_v, rows0, rows1, gsem0, gsem1, ssem0, ssem1):
    wid = lax.axis_index("s") * NC + lax.axis_index("c")
    base = wid * RPW

    pltpu.sync_copy(x_hbm.at[pl.ds(base, RPW)], idx_v)
    pltpu.sync_copy(pos_hbm, pos_v)

    bufs = (rows0, rows1)
    gsems = (gsem0, gsem1)
    ssems = (ssem0, ssem1)

    def start_gather(c, buf, sem):
        pltpu.async_copy(tok_hbm.at[idx_v.at[pl.ds(c * K, K)]], buf, sem)

    def start_scatter(c, buf, sem):
        pltpu.async_copy(buf, out_hbm.at[pl.ds(base + c * K, K)], sem)

    def wait_gather(c, buf, sem):
        pltpu.make_async_copy(tok_hbm.at[idx_v.at[pl.ds(c * K, K)]], buf, sem).wait()

    def wait_scatter(c, buf, sem):
        pltpu.make_async_copy(buf, out_hbm.at[pl.ds(base + c * K, K)], sem).wait()

    start_gather(0, bufs[0], gsems[0])

    def pair(p, carry):
        for b in range(2):
            c = 2 * p + b
            buf, nbuf = bufs[b], bufs[1 - b]

            @pl.when(c >= 1)
            def _():
                wait_scatter(c - 1, nbuf, ssems[1 - b])

            @pl.when(c + 1 < NCHUNK)
            def _():
                start_gather(c + 1, nbuf, gsems[1 - b])

            wait_gather(c, buf, gsems[b])

            s0 = lax.rem(c * K, S)

            def radd(r, carry2):
                sr = s0 + r
                s = lax.select(sr >= S, sr - S, sr)
                for j in range(D // LANES):
                    sl = pl.ds(j * LANES, LANES)
                    buf[r, sl] = buf[r, sl] + pos_v[s, sl]
                return carry2

            lax.fori_loop(0, K, radd, 0)
            start_scatter(c, buf, ssems[b])
        return carry

    lax.fori_loop(0, NCHUNK // 2, pair, 0)
    wait_scatter(NCHUNK - 1, bufs[1], ssems[1])


@functools.partial(
    pl.kernel,
    out_type=jax.ShapeDtypeStruct((N, D), jnp.float32),
    mesh=plsc.VectorSubcoreMesh(
        core_axis_name="c", subcore_axis_name="s", num_cores=NC, num_subcores=NS
    ),
    scratch_types=[
        pltpu.VMEM((RPW,), jnp.int32),
        pltpu.VMEM((S, D), jnp.float32),
        pltpu.VMEM((K, D), jnp.float32),
        pltpu.VMEM((K, D), jnp.float32),
        pltpu.SemaphoreType.DMA,
        pltpu.SemaphoreType.DMA,
        pltpu.SemaphoreType.DMA,
        pltpu.SemaphoreType.DMA,
    ],
)
def _emb(x_hbm, tok_hbm, pos_hbm, out_hbm,
         idx_v, pos_v, rows0, rows1, gsem0, gsem1, ssem0, ssem1):
    _emb_body(x_hbm, tok_hbm, pos_hbm, out_hbm,
              idx_v, pos_v, rows0, rows1, gsem0, gsem1, ssem0, ssem1)


def kernel(x, token_table, position_table):
    out = _emb(x.astype(jnp.int32).reshape(N), token_table, position_table)
    return out.reshape(B, S, D)


# speedup vs baseline: 1.8071x; 1.6415x over previous
import functools

import jax
import jax.numpy as jnp
from jax import lax
from jax.experimental import pallas as pl
from jax.experimental.pallas import tpu as pltpu
from jax.experimental.pallas import tpu_sc as plsc

B, S, D = 4096, 77, 512
N = B * S
NC, NS = 2, 16
NW = NC * NS
RPW = N // NW
K = 64
NCHUNK = RPW // K
LANES = 16


def _emb_body(x_hbm, tok_hbm, pos_hbm, out_hbm,
              idx_v, pos_v, rows0, rows1, gsem0, gsem1, ssem0, ssem1):
    wid = lax.axis_index("s") * NC + lax.axis_index("c")
    base = wid * RPW

    pltpu.sync_copy(x_hbm.at[pl.ds(base, RPW)], idx_v)
    pltpu.sync_copy(pos_hbm, pos_v)

    bufs = (rows0, rows1)
    gsems = (gsem0, gsem1)
    ssems = (ssem0, ssem1)

    def start_gather(c, buf, sem):
        pltpu.async_copy(tok_hbm.at[idx_v.at[pl.ds(c * K, K)]], buf, sem)

    def start_scatter(c, buf, sem):
        pltpu.async_copy(buf, out_hbm.at[pl.ds(base + c * K, K)], sem)

    def wait_gather(c, buf, sem):
        pltpu.make_async_copy(tok_hbm.at[idx_v.at[pl.ds(c * K, K)]], buf, sem).wait()

    def wait_scatter(c, buf, sem):
        pltpu.make_async_copy(buf, out_hbm.at[pl.ds(base + c * K, K)], sem).wait()

    start_gather(0, bufs[0], gsems[0])

    def pair(p, carry):
        for b in range(2):
            c = 2 * p + b
            buf, nbuf = bufs[b], bufs[1 - b]

            @pl.when(c >= 1)
            def _():
                wait_scatter(c - 1, nbuf, ssems[1 - b])

            @pl.when(c + 1 < NCHUNK)
            def _():
                start_gather(c + 1, nbuf, gsems[1 - b])

            wait_gather(c, buf, gsems[b])

            s0 = lax.rem(c * K, S)

            @plsc.parallel_loop(0, K, unroll=4)
            def _(r):
                sr = s0 + r
                s = lax.select(sr >= S, sr - S, sr)
                for j in range(D // LANES):
                    sl = pl.ds(j * LANES, LANES)
                    buf[r, sl] = buf[r, sl] + pos_v[s, sl]
            start_scatter(c, buf, ssems[b])
        return carry

    lax.fori_loop(0, NCHUNK // 2, pair, 0)
    wait_scatter(NCHUNK - 1, bufs[1], ssems[1])


@functools.partial(
    pl.kernel,
    out_type=jax.ShapeDtypeStruct((N, D), jnp.float32),
    mesh=plsc.VectorSubcoreMesh(
        core_axis_name="c", subcore_axis_name="s", num_cores=NC, num_subcores=NS
    ),
    scratch_types=[
        pltpu.VMEM((RPW,), jnp.int32),
        pltpu.VMEM((S, D), jnp.float32),
        pltpu.VMEM((K, D), jnp.float32),
        pltpu.VMEM((K, D), jnp.float32),
        pltpu.SemaphoreType.DMA,
        pltpu.SemaphoreType.DMA,
        pltpu.SemaphoreType.DMA,
        pltpu.SemaphoreType.DMA,
    ],
)
def _emb(x_hbm, tok_hbm, pos_hbm, out_hbm,
         idx_v, pos_v, rows0, rows1, gsem0, gsem1, ssem0, ssem1):
    _emb_body(x_hbm, tok_hbm, pos_hbm, out_hbm,
              idx_v, pos_v, rows0, rows1, gsem0, gsem1, ssem0, ssem1)


def kernel(x, token_table, position_table):
    out = _emb(x.astype(jnp.int32).reshape(N), token_table, position_table)
    return out.reshape(B, S, D)
